# Initial kernel scaffold; baseline (speedup 1.0000x reference)
import jax
import jax.numpy as jnp
from jax.experimental import pallas as pl


def kernel(x, edge_index, W1, b1, W2, b2, W3, b3, gamma1, beta1, gamma2, beta2):
    raise NotImplementedError("write your pallas kernel here")


# speedup vs baseline: 10.5151x; 10.5151x over previous
import functools

import jax
import jax.numpy as jnp
from jax import lax
from jax.experimental import pallas as pl
from jax.experimental.pallas import tpu as pltpu
from jax.experimental.pallas import tpu_sc as plsc

N = 10000
D = 128
E = 320000

NC = 2
NS = 16
NW = NC * NS
B = 128
CH = (E + NW * B - 1) // (NW * B)
EPT = CH * B
E_PAD = NW * EPT
NPAD = 10112
RPT = NPAD // NS
DDEG = 128


@functools.cache
def _sc_kernels():
    mesh = plsc.VectorSubcoreMesh(core_axis_name="c", subcore_axis_name="s",
                                  num_cores=NC, num_subcores=NS)

    @functools.partial(
        pl.kernel,
        out_type=jax.ShapeDtypeStruct((NC, NPAD, DDEG), jnp.float32),
        mesh=mesh,
        scratch_types=[
            pltpu.VMEM((CH, B), jnp.int32),
            pltpu.VMEM((B, DDEG), jnp.float32),
            pltpu.VMEM_SHARED((NPAD, DDEG), jnp.float32),
        ],
    )
    def deg_kernel(dst_hbm, ones_hbm, zeros_hbm, out_hbm, dst_v, ones_v, acc):
        c = lax.axis_index("c")
        s = lax.axis_index("s")
        wid = s * NC + c
        pltpu.sync_copy(zeros_hbm, acc.at[pl.ds(s * RPT, RPT)])
        pltpu.sync_copy(dst_hbm.at[wid], dst_v)
        pltpu.sync_copy(ones_hbm, ones_v)
        plsc.subcore_barrier()

        def body(j, carry):
            pltpu.sync_copy(ones_v, acc.at[dst_v.at[j]], add=True)
            return carry

        lax.fori_loop(0, CH, body, 0)
        plsc.subcore_barrier()
        pltpu.sync_copy(acc.at[pl.ds(s * RPT, RPT)],
                        out_hbm.at[c, pl.ds(s * RPT, RPT)])

    @functools.partial(
        pl.kernel,
        out_type=jax.ShapeDtypeStruct((NC, NPAD, D), jnp.float32),
        mesh=mesh,
        scratch_types=[
            pltpu.VMEM((CH, B), jnp.int32),
            pltpu.VMEM((CH, B), jnp.int32),
            pltpu.VMEM((B, D), jnp.float32),
            pltpu.VMEM_SHARED((NPAD, D), jnp.float32),
            pltpu.SemaphoreType.DMA,
        ],
    )
    def agg_kernel(src_hbm, dst_hbm, hs_hbm, zeros_hbm, out_hbm,
                   src_v, dst_v, rows_v, acc, sem):
        c = lax.axis_index("c")
        s = lax.axis_index("s")
        wid = s * NC + c
        pltpu.sync_copy(zeros_hbm, acc.at[pl.ds(s * RPT, RPT)])
        pltpu.sync_copy(src_hbm.at[wid], src_v)
        pltpu.sync_copy(dst_hbm.at[wid], dst_v)
        plsc.subcore_barrier()

        def body(j, carry):
            pltpu.async_copy(hs_hbm.at[src_v.at[j]], rows_v, sem).wait()
            pltpu.sync_copy(rows_v, acc.at[dst_v.at[j]], add=True)
            return carry

        lax.fori_loop(0, CH, body, 0)
        plsc.subcore_barrier()
        pltpu.sync_copy(acc.at[pl.ds(s * RPT, RPT)],
                        out_hbm.at[c, pl.ds(s * RPT, RPT)])

    return deg_kernel, agg_kernel


def _tc_stage1(x, W1, degP):
    def body(x_ref, w_ref, degp_ref, dinv_ref, h_ref, hs_ref):
        deg = degp_ref[0, :N, 0:1] + degp_ref[1, :N, 0:1] + 1.0
        dinv = lax.rsqrt(deg)
        h = jnp.dot(x_ref[...], w_ref[...], preferred_element_type=jnp.float32)
        dinv_ref[...] = dinv
        h_ref[...] = h
        hs_ref[...] = h * dinv

    return pl.pallas_call(
        body,
        out_shape=[
            jax.ShapeDtypeStruct((N, 1), jnp.float32),
            jax.ShapeDtypeStruct((N, D), jnp.float32),
            jax.ShapeDtypeStruct((N, D), jnp.float32),
        ],
    )(x, W1, degP)


def _tc_mid(S, h, dinv, b, gamma, beta, W_next):
    def body(s_ref, h_ref, dinv_ref, b_ref, g_ref, be_ref, w_ref,
             h2_ref, hs2_ref):
        dinv = dinv_ref[...]
        h = h_ref[...]
        agg = s_ref[0, :N, :] + s_ref[1, :N, :]
        conv = dinv * agg + (dinv * dinv) * h + b_ref[...]
        mean = jnp.mean(conv, axis=0, keepdims=True)
        var = jnp.mean((conv - mean) ** 2, axis=0, keepdims=True)
        y = g_ref[...] * (conv - mean) * lax.rsqrt(var + 1e-5) + be_ref[...]
        y = jnp.maximum(y, 0.0)
        h2 = jnp.dot(y, w_ref[...], preferred_element_type=jnp.float32)
        h2_ref[...] = h2
        hs2_ref[...] = h2 * dinv

    return pl.pallas_call(
        body,
        out_shape=[
            jax.ShapeDtypeStruct((N, D), jnp.float32),
            jax.ShapeDtypeStruct((N, D), jnp.float32),
        ],
    )(S, h, dinv, b.reshape(1, D), gamma.reshape(1, D), beta.reshape(1, D),
      W_next)


def _tc_final(S, h, dinv, b):
    def body(s_ref, h_ref, dinv_ref, b_ref, out_ref):
        dinv = dinv_ref[...]
        agg = s_ref[0, :N, :] + s_ref[1, :N, :]
        conv = dinv * agg + (dinv * dinv) * h_ref[...] + b_ref[...]
        m = jnp.max(conv, axis=-1, keepdims=True)
        z = conv - m
        lse = jnp.log(jnp.sum(jnp.exp(z), axis=-1, keepdims=True))
        out_ref[...] = z - lse

    return pl.pallas_call(
        body,
        out_shape=jax.ShapeDtypeStruct((N, D), jnp.float32),
    )(S, h, dinv, b.reshape(1, D))


def kernel(x, edge_index, W1, b1, W2, b2, W3, b3, gamma1, beta1,
           gamma2, beta2):
    src = edge_index[0].astype(jnp.int32)
    dst = edge_index[1].astype(jnp.int32)
    pad = E_PAD - E
    src3 = jnp.concatenate([src, jnp.zeros((pad,), jnp.int32)]
                           ).reshape(NW, CH, B)
    dst3 = jnp.concatenate([dst, jnp.full((pad,), N, jnp.int32)]
                           ).reshape(NW, CH, B)

    ones16 = jnp.ones((B, DDEG), jnp.float32)
    zeros16 = jnp.zeros((RPT, DDEG), jnp.float32)
    zeros128 = jnp.zeros((RPT, D), jnp.float32)

    deg_kernel, agg_kernel = _sc_kernels()
    degP = deg_kernel(dst3, ones16, zeros16)
    dinv, h1, hs1 = _tc_stage1(x, W1, degP)
    S1 = agg_kernel(src3, dst3, hs1, zeros128)
    h2, hs2 = _tc_mid(S1, h1, dinv, b1, gamma1, beta1, W2)
    S2 = agg_kernel(src3, dst3, hs2, zeros128)
    h3, hs3 = _tc_mid(S2, h2, dinv, b2, gamma2, beta2, W3)
    S3 = agg_kernel(src3, dst3, hs3, zeros128)
    return _tc_final(S3, h3, dinv, b3)
